# baseline (speedup 1.0000x reference)
import functools

import jax
import jax.numpy as jnp
from jax.experimental import pallas as pl
from jax.experimental.pallas import tpu as pltpu
from jax.experimental.pallas import tpu_sc as plsc

N = 10000
E_RAW = 320000
F_IN = 128
HID = 128
HEADS = 4
OUT_H = 32
NCLS = 16
NGRAPH = 64

BLK = 2000
NBLK = N // BLK

N_PAD = 10240
RPT = N_PAD // 16
C_E = 64
EC = 162
NW = 32
E_PAD = NW * EC * C_E

_PREC = jax.lax.Precision.HIGHEST


def _mm(a, b):
    return jax.lax.dot_general(a, b, (((1,), (0,)), ((), ())), precision=_PREC)


def _seg_matrix():
    r = jax.lax.broadcasted_iota(jnp.int32, (HID, 16), 0)
    c = jax.lax.broadcasted_iota(jnp.int32, (HID, 16), 1)
    return (c == r // OUT_H).astype(jnp.float32)


def _seg_matrix_t():
    r = jax.lax.broadcasted_iota(jnp.int32, (16, HID), 0)
    c = jax.lax.broadcasted_iota(jnp.int32, (16, HID), 1)
    return (r == c // OUT_H).astype(jnp.float32)


def _bn_scale_shift(stats_ref, w, b):
    m = stats_ref[0:1, :] * (1.0 / N)
    v = stats_ref[1:2, :] * (1.0 / N) - m * m
    scale = w * jax.lax.rsqrt(v + 1e-5)
    return scale, b - m * scale


def _finish_gat(num, den16, bias):
    denb = _mm(den16, _seg_matrix_t())
    return jnp.maximum(num / (denb + 1e-16) + bias, 0.0)


def _stats_body(x_ref, o_ref):
    @pl.when(pl.program_id(0) == 0)
    def _():
        o_ref[...] = jnp.zeros_like(o_ref)
    x = x_ref[...]
    o_ref[0:1, :] += jnp.sum(x, axis=0, keepdims=True)
    o_ref[1:2, :] += jnp.sum(x * x, axis=0, keepdims=True)


def _stats_call(x):
    return pl.pallas_call(
        _stats_body,
        grid=(NBLK,),
        in_specs=[pl.BlockSpec((BLK, HID), lambda i: (i, 0))],
        out_specs=pl.BlockSpec((2, HID), lambda i: (0, 0)),
        out_shape=jax.ShapeDtypeStruct((2, HID), jnp.float32),
    )(x)


def _inproj_body(stats_ref, x_ref, w0_ref, b0_ref, bnw_ref, bnb_ref,
                 z_ref, zst_ref):
    scale, shift = _bn_scale_shift(stats_ref, bnw_ref[...], bnb_ref[...])
    z = jnp.maximum(_mm(x_ref[...] * scale, w0_ref[...])
                    + _mm(shift, w0_ref[...]) + b0_ref[...], 0.0)
    z_ref[...] = z
    @pl.when(pl.program_id(0) == 0)
    def _():
        zst_ref[...] = jnp.zeros_like(zst_ref)
    zst_ref[0:1, :] += jnp.sum(z, axis=0, keepdims=True)
    zst_ref[1:2, :] += jnp.sum(z * z, axis=0, keepdims=True)


def _inproj_call(stats, x, w0, b0, bnw, bnb):
    full = lambda r: pl.BlockSpec((r, HID), lambda i: (0, 0))
    return pl.pallas_call(
        _inproj_body,
        grid=(NBLK,),
        in_specs=[full(2), pl.BlockSpec((BLK, HID), lambda i: (i, 0)),
                  full(HID), full(1), full(1), full(1)],
        out_specs=(pl.BlockSpec((BLK, HID), lambda i: (i, 0)),
                   pl.BlockSpec((2, HID), lambda i: (0, 0))),
        out_shape=(jax.ShapeDtypeStruct((N, HID), jnp.float32),
                   jax.ShapeDtypeStruct((2, HID), jnp.float32)),
    )(stats, x, w0, b0, bnw, bnb)


def _gatproj_body(stats_ref, z_ref, w_ref, bnw_ref, bnb_ref, af_s_ref,
                  af_d_ref, h_ref, as_ref, ad_ref):
    scale, shift = _bn_scale_shift(stats_ref, bnw_ref[...], bnb_ref[...])
    h = _mm(z_ref[...] * scale, w_ref[...]) + _mm(shift, w_ref[...])
    h_ref[...] = h
    s = _seg_matrix()
    as_ref[...] = _mm(h * af_s_ref[...], s)
    ad_ref[...] = _mm(h * af_d_ref[...], s)


def _gatproj_call(stats, z, w, bnw, bnb, af_s, af_d):
    full = lambda r: pl.BlockSpec((r, HID), lambda i: (0, 0))
    return pl.pallas_call(
        _gatproj_body,
        grid=(NBLK,),
        in_specs=[full(2), pl.BlockSpec((BLK, HID), lambda i: (i, 0)),
                  full(HID), full(1), full(1), full(1), full(1)],
        out_specs=(pl.BlockSpec((BLK, HID), lambda i: (i, 0)),
                   pl.BlockSpec((BLK, 16), lambda i: (i, 0)),
                   pl.BlockSpec((BLK, 16), lambda i: (i, 0))),
        out_shape=(jax.ShapeDtypeStruct((N, HID), jnp.float32),
                   jax.ShapeDtypeStruct((N, 16), jnp.float32),
                   jax.ShapeDtypeStruct((N, 16), jnp.float32)),
    )(stats, z, w, bnw, bnb, af_s, af_d)


def _gatfin_body(numa_ref, numb_ref, dena_ref, denb_ref, bias_ref,
                 o_ref, ost_ref):
    num = numa_ref[0] + numb_ref[0]
    den16 = dena_ref[0] + denb_ref[0]
    o = _finish_gat(num, den16, bias_ref[...])
    o_ref[...] = o
    @pl.when(pl.program_id(0) == 0)
    def _():
        ost_ref[...] = jnp.zeros_like(ost_ref)
    ost_ref[0:1, :] += jnp.sum(o, axis=0, keepdims=True)
    ost_ref[1:2, :] += jnp.sum(o * o, axis=0, keepdims=True)


def _gatfin_call(num2, den2, bias):
    full = lambda r: pl.BlockSpec((r, HID), lambda i: (0, 0))
    return pl.pallas_call(
        _gatfin_body,
        grid=(NBLK,),
        in_specs=[pl.BlockSpec((1, BLK, HID), lambda i: (0, i, 0)),
                  pl.BlockSpec((1, BLK, HID), lambda i: (1, i, 0)),
                  pl.BlockSpec((1, BLK, 16), lambda i: (0, i, 0)),
                  pl.BlockSpec((1, BLK, 16), lambda i: (1, i, 0)), full(1)],
        out_specs=(pl.BlockSpec((BLK, HID), lambda i: (i, 0)),
                   pl.BlockSpec((2, HID), lambda i: (0, 0))),
        out_shape=(jax.ShapeDtypeStruct((N, HID), jnp.float32),
                   jax.ShapeDtypeStruct((2, HID), jnp.float32)),
    )(num2, num2, den2, den2, bias)


def _pool_body(numa_ref, numb_ref, dena_ref, denb_ref, bias_ref, batch_ref,
               g_ref):
    num = numa_ref[0] + numb_ref[0]
    den16 = dena_ref[0] + denb_ref[0]
    o = _finish_gat(num, den16, bias_ref[...])
    gids = jax.lax.broadcasted_iota(jnp.int32, (NGRAPH, BLK), 0)
    m = (gids == batch_ref[0]).astype(jnp.float32)
    @pl.when(pl.program_id(0) == 0)
    def _():
        g_ref[...] = jnp.zeros_like(g_ref)
    g_ref[...] += _mm(m, o)


def _pool_call(num2, den2, bias, batch3d):
    full = lambda r: pl.BlockSpec((r, HID), lambda i: (0, 0))
    return pl.pallas_call(
        _pool_body,
        grid=(NBLK,),
        in_specs=[pl.BlockSpec((1, BLK, HID), lambda i: (0, i, 0)),
                  pl.BlockSpec((1, BLK, HID), lambda i: (1, i, 0)),
                  pl.BlockSpec((1, BLK, 16), lambda i: (0, i, 0)),
                  pl.BlockSpec((1, BLK, 16), lambda i: (1, i, 0)), full(1),
                  pl.BlockSpec((1, 1, BLK), lambda i: (i, 0, 0))],
        out_specs=pl.BlockSpec((NGRAPH, HID), lambda i: (0, 0)),
        out_shape=jax.ShapeDtypeStruct((NGRAPH, HID), jnp.float32),
    )(num2, num2, den2, den2, bias, batch3d)


def _edge_body(h_hbm, as_hbm, ad_hbm, src_hbm, dst_hbm, zrow_hbm, zden_hbm,
               num_out, den_out,
               num_acc, den_acc, src_v, dst_v, arows, brows, wrows, hrows,
               sems):
    cid = jax.lax.axis_index("c")
    sid = jax.lax.axis_index("s")
    base_row = sid * RPT
    pltpu.sync_copy(zrow_hbm, num_acc.at[pl.ds(base_row, RPT), :])
    pltpu.sync_copy(zden_hbm, den_acc.at[pl.ds(base_row, RPT), :])
    plsc.subcore_barrier()
    wid = cid * 16 + sid

    def issue(b, base):
        pltpu.sync_copy(src_hbm.at[pl.ds(base, C_E)], src_v.at[b])
        pltpu.sync_copy(dst_hbm.at[pl.ds(base, C_E)], dst_v.at[b])
        pltpu.async_copy(h_hbm.at[src_v.at[b]], hrows.at[b], sems.at[b, 0])
        pltpu.async_copy(as_hbm.at[src_v.at[b]], arows.at[b], sems.at[b, 1])
        pltpu.async_copy(ad_hbm.at[dst_v.at[b]], brows.at[b], sems.at[b, 2])

    def process(b):
        pltpu.make_async_copy(as_hbm.at[src_v.at[b]], arows.at[b],
                              sems.at[b, 1]).wait()
        pltpu.make_async_copy(ad_hbm.at[dst_v.at[b]], brows.at[b],
                              sems.at[b, 2]).wait()
        pltpu.make_async_copy(h_hbm.at[src_v.at[b]], hrows.at[b],
                              sems.at[b, 0]).wait()

        def eloop(e, c2):
            s16 = arows[b, e, :] + brows[b, e, :]
            w16 = jnp.exp(jnp.maximum(s16, 0.0) + 0.2 * jnp.minimum(s16, 0.0))
            wrows[b, e, :] = w16
            for hd in range(HEADS):
                ws = w16[hd]
                for q in range(2):
                    col = hd * OUT_H + q * 16
                    hrows[b, e, pl.ds(col, 16)] = (
                        hrows[b, e, pl.ds(col, 16)] * ws)
            return c2
        jax.lax.fori_loop(0, C_E, eloop, 0)
        pltpu.sync_copy(wrows.at[b], den_acc.at[dst_v.at[b]], add=True)
        pltpu.sync_copy(hrows.at[b], num_acc.at[dst_v.at[b]], add=True)

    cbase = wid * EC * C_E
    issue(0, cbase)

    def pair(j, carry):
        base_a = cbase + (2 * j) * C_E
        issue(1, base_a + C_E)
        process(0)
        @pl.when(2 * j + 2 < EC)
        def _():
            issue(0, base_a + 2 * C_E)
        process(1)
        return carry

    jax.lax.fori_loop(0, EC // 2, pair, 0)
    plsc.subcore_barrier()
    pltpu.sync_copy(num_acc.at[pl.ds(base_row, RPT), :],
                    num_out.at[cid, pl.ds(base_row, RPT), :])
    pltpu.sync_copy(den_acc.at[pl.ds(base_row, RPT), :],
                    den_out.at[cid, pl.ds(base_row, RPT), :])


_edge_call = pl.kernel(
    _edge_body,
    out_type=(jax.ShapeDtypeStruct((2, N_PAD, HID), jnp.float32),
              jax.ShapeDtypeStruct((2, N_PAD, 16), jnp.float32)),
    mesh=plsc.VectorSubcoreMesh(core_axis_name="c", subcore_axis_name="s"),
    compiler_params=pltpu.CompilerParams(use_tc_tiling_on_sc=False),
    scratch_types=[
        pltpu.VMEM_SHARED((N_PAD, HID), jnp.float32),
        pltpu.VMEM_SHARED((N_PAD, 16), jnp.float32),
        pltpu.VMEM((2, C_E), jnp.int32),
        pltpu.VMEM((2, C_E), jnp.int32),
        pltpu.VMEM((2, C_E, 16), jnp.float32),
        pltpu.VMEM((2, C_E, 16), jnp.float32),
        pltpu.VMEM((2, C_E, 16), jnp.float32),
        pltpu.VMEM((2, C_E, HID), jnp.float32),
        pltpu.SemaphoreType.DMA((2, 3)),
    ],
)


def _edge_aggregate(h_pad, as_pad, ad_pad, src_pad, dst_pad):
    zrow = jnp.zeros((RPT, HID), jnp.float32)
    zden = jnp.zeros((RPT, 16), jnp.float32)
    return _edge_call(h_pad, as_pad, ad_pad, src_pad, dst_pad, zrow, zden)


def _head_body(g_ref, bnw_ref, bnb_ref, wc_ref, bc_ref, out_ref):
    g = g_ref[...]
    m = jnp.mean(g, axis=0, keepdims=True)
    v = jnp.mean((g - m) ** 2, axis=0, keepdims=True)
    g = (g - m) * jax.lax.rsqrt(v + 1e-5) * bnw_ref[...] + bnb_ref[...]
    logits = _mm(g, wc_ref[...]) + bc_ref[...]
    mx = jnp.max(logits, axis=-1, keepdims=True)
    lse = mx + jnp.log(jnp.sum(jnp.exp(logits - mx), axis=-1, keepdims=True))
    out_ref[...] = logits - lse


def _head_call(g, bnw, bnb, wc, bc):
    return pl.pallas_call(
        _head_body,
        out_shape=jax.ShapeDtypeStruct((NGRAPH, NCLS), jnp.float32),
    )(g, bnw, bnb, wc, bc)


def _padn(a):
    return jnp.pad(a, ((0, N_PAD - N), (0, 0)))


def kernel(x, edge_index, batch, bn_feat_w, bn_feat_b, W0, b0,
           bn1_w, bn1_b, W1, a1s, a1d, bias1,
           bn2_w, bn2_b, W2, a2s, a2d, bias2,
           bn3_w, bn3_b, W3, a3s, a3d, bias3,
           bnh_w, bnh_b, Wc, bc):
    r1 = lambda v: v.reshape(1, -1)
    loops = jnp.arange(N, dtype=jnp.int32)
    padidx = jnp.full((E_PAD - E_RAW - N,), N, jnp.int32)
    src = jnp.concatenate([edge_index[0], loops, padidx])
    dst = jnp.concatenate([edge_index[1], loops, padidx])

    xst = _stats_call(x)
    z1, zst = _inproj_call(xst, x, W0, r1(b0), r1(bn_feat_w), r1(bn_feat_b))
    h1, as1, ad1 = _gatproj_call(zst, z1, W1, r1(bn1_w), r1(bn1_b),
                                 r1(a1s), r1(a1d))
    num1, den1 = _edge_aggregate(_padn(h1), _padn(as1), _padn(ad1), src, dst)

    o1, o1st = _gatfin_call(num1, den1, r1(bias1))
    h2, as2, ad2 = _gatproj_call(o1st, o1, W2, r1(bn2_w), r1(bn2_b),
                                 r1(a2s), r1(a2d))
    num2, den2 = _edge_aggregate(_padn(h2), _padn(as2), _padn(ad2), src, dst)

    o2, o2st = _gatfin_call(num2, den2, r1(bias2))
    h3, as3, ad3 = _gatproj_call(o2st, o2, W3, r1(bn3_w), r1(bn3_b),
                                 r1(a3s), r1(a3d))
    num3, den3 = _edge_aggregate(_padn(h3), _padn(as3), _padn(ad3), src, dst)

    g = _pool_call(num3, den3, r1(bias3), batch.reshape(NBLK, 1, BLK))
    return _head_call(g, r1(bnh_w), r1(bnh_b), Wc, r1(bc))


# speedup vs baseline: 1.1689x; 1.1689x over previous
import functools

import jax
import jax.numpy as jnp
from jax.experimental import pallas as pl
from jax.experimental.pallas import tpu as pltpu
from jax.experimental.pallas import tpu_sc as plsc

N = 10000
E_RAW = 320000
F_IN = 128
HID = 128
HEADS = 4
OUT_H = 32
NCLS = 16
NGRAPH = 64

BLK = 2000
NBLK = N // BLK

N_PAD = 10240
RPT = N_PAD // 16
C_E = 64
EC = 162
NW = 32
E_PAD = NW * EC * C_E

_PREC = jax.lax.Precision.HIGHEST


def _mm(a, b):
    return jax.lax.dot_general(a, b, (((1,), (0,)), ((), ())), precision=_PREC)


def _seg_matrix():
    r = jax.lax.broadcasted_iota(jnp.int32, (HID, 16), 0)
    c = jax.lax.broadcasted_iota(jnp.int32, (HID, 16), 1)
    return (c == r // OUT_H).astype(jnp.float32)


def _seg_matrix_t():
    r = jax.lax.broadcasted_iota(jnp.int32, (16, HID), 0)
    c = jax.lax.broadcasted_iota(jnp.int32, (16, HID), 1)
    return (r == c // OUT_H).astype(jnp.float32)


def _bn_scale_shift(stats_ref, w, b):
    m = stats_ref[0:1, :] * (1.0 / N)
    v = stats_ref[1:2, :] * (1.0 / N) - m * m
    scale = w * jax.lax.rsqrt(v + 1e-5)
    return scale, b - m * scale


def _finish_gat(num, den16, bias):
    denb = _mm(den16, _seg_matrix_t())
    return jnp.maximum(num / (denb + 1e-16) + bias, 0.0)


def _stats_body(x_ref, o_ref):
    @pl.when(pl.program_id(0) == 0)
    def _():
        o_ref[...] = jnp.zeros_like(o_ref)
    x = x_ref[...]
    o_ref[0:1, :] += jnp.sum(x, axis=0, keepdims=True)
    o_ref[1:2, :] += jnp.sum(x * x, axis=0, keepdims=True)


def _stats_call(x):
    return pl.pallas_call(
        _stats_body,
        grid=(NBLK,),
        in_specs=[pl.BlockSpec((BLK, HID), lambda i: (i, 0))],
        out_specs=pl.BlockSpec((2, HID), lambda i: (0, 0)),
        out_shape=jax.ShapeDtypeStruct((2, HID), jnp.float32),
    )(x)


def _inproj_body(stats_ref, x_ref, w0_ref, b0_ref, bnw_ref, bnb_ref,
                 z_ref, zst_ref):
    scale, shift = _bn_scale_shift(stats_ref, bnw_ref[...], bnb_ref[...])
    z = jnp.maximum(_mm(x_ref[...] * scale, w0_ref[...])
                    + _mm(shift, w0_ref[...]) + b0_ref[...], 0.0)
    z_ref[...] = z
    @pl.when(pl.program_id(0) == 0)
    def _():
        zst_ref[...] = jnp.zeros_like(zst_ref)
    zst_ref[0:1, :] += jnp.sum(z, axis=0, keepdims=True)
    zst_ref[1:2, :] += jnp.sum(z * z, axis=0, keepdims=True)


def _inproj_call(stats, x, w0, b0, bnw, bnb):
    full = lambda r: pl.BlockSpec((r, HID), lambda i: (0, 0))
    return pl.pallas_call(
        _inproj_body,
        grid=(NBLK,),
        in_specs=[full(2), pl.BlockSpec((BLK, HID), lambda i: (i, 0)),
                  full(HID), full(1), full(1), full(1)],
        out_specs=(pl.BlockSpec((BLK, HID), lambda i: (i, 0)),
                   pl.BlockSpec((2, HID), lambda i: (0, 0))),
        out_shape=(jax.ShapeDtypeStruct((N, HID), jnp.float32),
                   jax.ShapeDtypeStruct((2, HID), jnp.float32)),
    )(stats, x, w0, b0, bnw, bnb)


def _gatproj_body(stats_ref, z_ref, w_ref, bnw_ref, bnb_ref, af_s_ref,
                  af_d_ref, h_ref, as_ref, ad_ref):
    scale, shift = _bn_scale_shift(stats_ref, bnw_ref[...], bnb_ref[...])
    h = _mm(z_ref[...] * scale, w_ref[...]) + _mm(shift, w_ref[...])
    h_ref[...] = h
    s = _seg_matrix()
    as_ref[...] = _mm(h * af_s_ref[...], s)
    ad_ref[...] = _mm(h * af_d_ref[...], s)


def _gatproj_call(stats, z, w, bnw, bnb, af_s, af_d):
    full = lambda r: pl.BlockSpec((r, HID), lambda i: (0, 0))
    return pl.pallas_call(
        _gatproj_body,
        grid=(NBLK,),
        in_specs=[full(2), pl.BlockSpec((BLK, HID), lambda i: (i, 0)),
                  full(HID), full(1), full(1), full(1), full(1)],
        out_specs=(pl.BlockSpec((BLK, HID), lambda i: (i, 0)),
                   pl.BlockSpec((BLK, 16), lambda i: (i, 0)),
                   pl.BlockSpec((BLK, 16), lambda i: (i, 0))),
        out_shape=(jax.ShapeDtypeStruct((N, HID), jnp.float32),
                   jax.ShapeDtypeStruct((N, 16), jnp.float32),
                   jax.ShapeDtypeStruct((N, 16), jnp.float32)),
    )(stats, z, w, bnw, bnb, af_s, af_d)


def _gatfin_body(numa_ref, numb_ref, dena_ref, denb_ref, bias_ref,
                 o_ref, ost_ref):
    num = numa_ref[0] + numb_ref[0]
    den16 = dena_ref[0] + denb_ref[0]
    o = _finish_gat(num, den16, bias_ref[...])
    o_ref[...] = o
    @pl.when(pl.program_id(0) == 0)
    def _():
        ost_ref[...] = jnp.zeros_like(ost_ref)
    ost_ref[0:1, :] += jnp.sum(o, axis=0, keepdims=True)
    ost_ref[1:2, :] += jnp.sum(o * o, axis=0, keepdims=True)


def _gatfin_call(num2, den2, bias):
    full = lambda r: pl.BlockSpec((r, HID), lambda i: (0, 0))
    return pl.pallas_call(
        _gatfin_body,
        grid=(NBLK,),
        in_specs=[pl.BlockSpec((1, BLK, HID), lambda i: (0, i, 0)),
                  pl.BlockSpec((1, BLK, HID), lambda i: (1, i, 0)),
                  pl.BlockSpec((1, BLK, 16), lambda i: (0, i, 0)),
                  pl.BlockSpec((1, BLK, 16), lambda i: (1, i, 0)), full(1)],
        out_specs=(pl.BlockSpec((BLK, HID), lambda i: (i, 0)),
                   pl.BlockSpec((2, HID), lambda i: (0, 0))),
        out_shape=(jax.ShapeDtypeStruct((N, HID), jnp.float32),
                   jax.ShapeDtypeStruct((2, HID), jnp.float32)),
    )(num2, num2, den2, den2, bias)


def _pool_body(numa_ref, numb_ref, dena_ref, denb_ref, bias_ref, batch_ref,
               g_ref):
    num = numa_ref[0] + numb_ref[0]
    den16 = dena_ref[0] + denb_ref[0]
    o = _finish_gat(num, den16, bias_ref[...])
    gids = jax.lax.broadcasted_iota(jnp.int32, (NGRAPH, BLK), 0)
    m = (gids == batch_ref[0]).astype(jnp.float32)
    @pl.when(pl.program_id(0) == 0)
    def _():
        g_ref[...] = jnp.zeros_like(g_ref)
    g_ref[...] += _mm(m, o)


def _pool_call(num2, den2, bias, batch3d):
    full = lambda r: pl.BlockSpec((r, HID), lambda i: (0, 0))
    return pl.pallas_call(
        _pool_body,
        grid=(NBLK,),
        in_specs=[pl.BlockSpec((1, BLK, HID), lambda i: (0, i, 0)),
                  pl.BlockSpec((1, BLK, HID), lambda i: (1, i, 0)),
                  pl.BlockSpec((1, BLK, 16), lambda i: (0, i, 0)),
                  pl.BlockSpec((1, BLK, 16), lambda i: (1, i, 0)), full(1),
                  pl.BlockSpec((1, 1, BLK), lambda i: (i, 0, 0))],
        out_specs=pl.BlockSpec((NGRAPH, HID), lambda i: (0, 0)),
        out_shape=jax.ShapeDtypeStruct((NGRAPH, HID), jnp.float32),
    )(num2, num2, den2, den2, bias, batch3d)


def _edge_body(h_hbm, as_hbm, ad_hbm, src_hbm, dst_hbm, zrow_hbm, zden_hbm,
               num_out, den_out,
               num_acc, den_acc, src_v, dst_v, arows, brows, wrows, hrows,
               sems):
    cid = jax.lax.axis_index("c")
    sid = jax.lax.axis_index("s")
    base_row = sid * RPT
    pltpu.sync_copy(zrow_hbm, num_acc.at[pl.ds(base_row, RPT), :])
    pltpu.sync_copy(zden_hbm, den_acc.at[pl.ds(base_row, RPT), :])
    plsc.subcore_barrier()
    wid = cid * 16 + sid

    def issue(b, base):
        pltpu.sync_copy(src_hbm.at[pl.ds(base, C_E)], src_v.at[b])
        pltpu.sync_copy(dst_hbm.at[pl.ds(base, C_E)], dst_v.at[b])
        pltpu.async_copy(h_hbm.at[src_v.at[b]], hrows.at[b], sems.at[b, 0])
        pltpu.async_copy(as_hbm.at[src_v.at[b]], arows.at[b], sems.at[b, 1])
        pltpu.async_copy(ad_hbm.at[dst_v.at[b]], brows.at[b], sems.at[b, 2])

    def process(b):
        pltpu.make_async_copy(as_hbm.at[src_v.at[b]], arows.at[b],
                              sems.at[b, 1]).wait()
        pltpu.make_async_copy(ad_hbm.at[dst_v.at[b]], brows.at[b],
                              sems.at[b, 2]).wait()
        pltpu.make_async_copy(h_hbm.at[src_v.at[b]], hrows.at[b],
                              sems.at[b, 0]).wait()

        def eloop(u, c2):
            e0 = u * 4
            ws_all = []
            for du in range(4):
                e = e0 + du
                s16 = arows[b, e, :] + brows[b, e, :]
                w16 = jnp.exp(jnp.maximum(s16, 0.0)
                              + 0.2 * jnp.minimum(s16, 0.0))
                wrows[b, e, :] = w16
                ws_all.append([w16[hd] for hd in range(HEADS)])
            for du in range(4):
                e = e0 + du
                for hd in range(HEADS):
                    ws = ws_all[du][hd]
                    for q in range(2):
                        col = hd * OUT_H + q * 16
                        hrows[b, e, pl.ds(col, 16)] = (
                            hrows[b, e, pl.ds(col, 16)] * ws)
            return c2
        jax.lax.fori_loop(0, C_E // 4, eloop, 0)
        pltpu.sync_copy(wrows.at[b], den_acc.at[dst_v.at[b]], add=True)
        pltpu.sync_copy(hrows.at[b], num_acc.at[dst_v.at[b]], add=True)

    cbase = wid * EC * C_E
    issue(0, cbase)

    def pair(j, carry):
        base_a = cbase + (2 * j) * C_E
        issue(1, base_a + C_E)
        process(0)
        @pl.when(2 * j + 2 < EC)
        def _():
            issue(0, base_a + 2 * C_E)
        process(1)
        return carry

    jax.lax.fori_loop(0, EC // 2, pair, 0)
    plsc.subcore_barrier()
    pltpu.sync_copy(num_acc.at[pl.ds(base_row, RPT), :],
                    num_out.at[cid, pl.ds(base_row, RPT), :])
    pltpu.sync_copy(den_acc.at[pl.ds(base_row, RPT), :],
                    den_out.at[cid, pl.ds(base_row, RPT), :])


_edge_call = pl.kernel(
    _edge_body,
    out_type=(jax.ShapeDtypeStruct((2, N_PAD, HID), jnp.float32),
              jax.ShapeDtypeStruct((2, N_PAD, 16), jnp.float32)),
    mesh=plsc.VectorSubcoreMesh(core_axis_name="c", subcore_axis_name="s"),
    compiler_params=pltpu.CompilerParams(use_tc_tiling_on_sc=False),
    scratch_types=[
        pltpu.VMEM_SHARED((N_PAD, HID), jnp.float32),
        pltpu.VMEM_SHARED((N_PAD, 16), jnp.float32),
        pltpu.VMEM((2, C_E), jnp.int32),
        pltpu.VMEM((2, C_E), jnp.int32),
        pltpu.VMEM((2, C_E, 16), jnp.float32),
        pltpu.VMEM((2, C_E, 16), jnp.float32),
        pltpu.VMEM((2, C_E, 16), jnp.float32),
        pltpu.VMEM((2, C_E, HID), jnp.float32),
        pltpu.SemaphoreType.DMA((2, 3)),
    ],
)


def _edge_aggregate(h_pad, as_pad, ad_pad, src_pad, dst_pad):
    zrow = jnp.zeros((RPT, HID), jnp.float32)
    zden = jnp.zeros((RPT, 16), jnp.float32)
    return _edge_call(h_pad, as_pad, ad_pad, src_pad, dst_pad, zrow, zden)


def _head_body(g_ref, bnw_ref, bnb_ref, wc_ref, bc_ref, out_ref):
    g = g_ref[...]
    m = jnp.mean(g, axis=0, keepdims=True)
    v = jnp.mean((g - m) ** 2, axis=0, keepdims=True)
    g = (g - m) * jax.lax.rsqrt(v + 1e-5) * bnw_ref[...] + bnb_ref[...]
    logits = _mm(g, wc_ref[...]) + bc_ref[...]
    mx = jnp.max(logits, axis=-1, keepdims=True)
    lse = mx + jnp.log(jnp.sum(jnp.exp(logits - mx), axis=-1, keepdims=True))
    out_ref[...] = logits - lse


def _head_call(g, bnw, bnb, wc, bc):
    return pl.pallas_call(
        _head_body,
        out_shape=jax.ShapeDtypeStruct((NGRAPH, NCLS), jnp.float32),
    )(g, bnw, bnb, wc, bc)


def _padn(a):
    return jnp.pad(a, ((0, N_PAD - N), (0, 0)))


def kernel(x, edge_index, batch, bn_feat_w, bn_feat_b, W0, b0,
           bn1_w, bn1_b, W1, a1s, a1d, bias1,
           bn2_w, bn2_b, W2, a2s, a2d, bias2,
           bn3_w, bn3_b, W3, a3s, a3d, bias3,
           bnh_w, bnh_b, Wc, bc):
    r1 = lambda v: v.reshape(1, -1)
    loops = jnp.arange(N, dtype=jnp.int32)
    padidx = jnp.full((E_PAD - E_RAW - N,), N, jnp.int32)
    src = jnp.concatenate([edge_index[0], loops, padidx])
    dst = jnp.concatenate([edge_index[1], loops, padidx])

    xst = _stats_call(x)
    z1, zst = _inproj_call(xst, x, W0, r1(b0), r1(bn_feat_w), r1(bn_feat_b))
    h1, as1, ad1 = _gatproj_call(zst, z1, W1, r1(bn1_w), r1(bn1_b),
                                 r1(a1s), r1(a1d))
    num1, den1 = _edge_aggregate(_padn(h1), _padn(as1), _padn(ad1), src, dst)

    o1, o1st = _gatfin_call(num1, den1, r1(bias1))
    h2, as2, ad2 = _gatproj_call(o1st, o1, W2, r1(bn2_w), r1(bn2_b),
                                 r1(a2s), r1(a2d))
    num2, den2 = _edge_aggregate(_padn(h2), _padn(as2), _padn(ad2), src, dst)

    o2, o2st = _gatfin_call(num2, den2, r1(bias2))
    h3, as3, ad3 = _gatproj_call(o2st, o2, W3, r1(bn3_w), r1(bn3_b),
                                 r1(a3s), r1(a3d))
    num3, den3 = _edge_aggregate(_padn(h3), _padn(as3), _padn(ad3), src, dst)

    g = _pool_call(num3, den3, r1(bias3), batch.reshape(NBLK, 1, BLK))
    return _head_call(g, r1(bnh_w), r1(bnh_b), Wc, r1(bc))


# speedup vs baseline: 1.1895x; 1.0176x over previous
import functools

import jax
import jax.numpy as jnp
from jax.experimental import pallas as pl
from jax.experimental.pallas import tpu as pltpu
from jax.experimental.pallas import tpu_sc as plsc

N = 10000
E_RAW = 320000
F_IN = 128
HID = 128
HEADS = 4
OUT_H = 32
NCLS = 16
NGRAPH = 64

BLK = 2000
NBLK = N // BLK

N_PAD = 10240
RPT = N_PAD // 16
C_E = 64
EC = 162
NW = 32
E_PAD = NW * EC * C_E

_PREC = jax.lax.Precision.HIGHEST


def _mm(a, b):
    return jax.lax.dot_general(a, b, (((1,), (0,)), ((), ())), precision=_PREC)


def _seg_matrix():
    r = jax.lax.broadcasted_iota(jnp.int32, (HID, 16), 0)
    c = jax.lax.broadcasted_iota(jnp.int32, (HID, 16), 1)
    return (c == r // OUT_H).astype(jnp.float32)


def _seg_matrix_t():
    r = jax.lax.broadcasted_iota(jnp.int32, (16, HID), 0)
    c = jax.lax.broadcasted_iota(jnp.int32, (16, HID), 1)
    return (r == c // OUT_H).astype(jnp.float32)


def _bn_scale_shift(stats_ref, w, b):
    m = stats_ref[0:1, :] * (1.0 / N)
    v = stats_ref[1:2, :] * (1.0 / N) - m * m
    scale = w * jax.lax.rsqrt(v + 1e-5)
    return scale, b - m * scale


def _finish_gat(num, den16, bias):
    denb = _mm(den16, _seg_matrix_t())
    return jnp.maximum(num / (denb + 1e-16) + bias, 0.0)


def _stats_body(x_ref, o_ref):
    @pl.when(pl.program_id(0) == 0)
    def _():
        o_ref[...] = jnp.zeros_like(o_ref)
    x = x_ref[...]
    o_ref[0:1, :] += jnp.sum(x, axis=0, keepdims=True)
    o_ref[1:2, :] += jnp.sum(x * x, axis=0, keepdims=True)


def _stats_call(x):
    return pl.pallas_call(
        _stats_body,
        grid=(NBLK,),
        in_specs=[pl.BlockSpec((BLK, HID), lambda i: (i, 0))],
        out_specs=pl.BlockSpec((2, HID), lambda i: (0, 0)),
        out_shape=jax.ShapeDtypeStruct((2, HID), jnp.float32),
    )(x)


def _inproj_body(stats_ref, x_ref, w0_ref, b0_ref, bnw_ref, bnb_ref,
                 z_ref, zst_ref):
    scale, shift = _bn_scale_shift(stats_ref, bnw_ref[...], bnb_ref[...])
    z = jnp.maximum(_mm(x_ref[...] * scale, w0_ref[...])
                    + _mm(shift, w0_ref[...]) + b0_ref[...], 0.0)
    z_ref[...] = z
    @pl.when(pl.program_id(0) == 0)
    def _():
        zst_ref[...] = jnp.zeros_like(zst_ref)
    zst_ref[0:1, :] += jnp.sum(z, axis=0, keepdims=True)
    zst_ref[1:2, :] += jnp.sum(z * z, axis=0, keepdims=True)


def _inproj_call(stats, x, w0, b0, bnw, bnb):
    full = lambda r: pl.BlockSpec((r, HID), lambda i: (0, 0))
    return pl.pallas_call(
        _inproj_body,
        grid=(NBLK,),
        in_specs=[full(2), pl.BlockSpec((BLK, HID), lambda i: (i, 0)),
                  full(HID), full(1), full(1), full(1)],
        out_specs=(pl.BlockSpec((BLK, HID), lambda i: (i, 0)),
                   pl.BlockSpec((2, HID), lambda i: (0, 0))),
        out_shape=(jax.ShapeDtypeStruct((N, HID), jnp.float32),
                   jax.ShapeDtypeStruct((2, HID), jnp.float32)),
    )(stats, x, w0, b0, bnw, bnb)


def _gatproj_body(stats_ref, z_ref, w_ref, bnw_ref, bnb_ref, af_s_ref,
                  af_d_ref, h_ref, as_ref, ad_ref):
    scale, shift = _bn_scale_shift(stats_ref, bnw_ref[...], bnb_ref[...])
    h = _mm(z_ref[...] * scale, w_ref[...]) + _mm(shift, w_ref[...])
    h_ref[...] = h
    s = _seg_matrix()
    as_ref[...] = _mm(h * af_s_ref[...], s)
    ad_ref[...] = _mm(h * af_d_ref[...], s)


def _gatproj_call(stats, z, w, bnw, bnb, af_s, af_d):
    full = lambda r: pl.BlockSpec((r, HID), lambda i: (0, 0))
    return pl.pallas_call(
        _gatproj_body,
        grid=(NBLK,),
        in_specs=[full(2), pl.BlockSpec((BLK, HID), lambda i: (i, 0)),
                  full(HID), full(1), full(1), full(1), full(1)],
        out_specs=(pl.BlockSpec((BLK, HID), lambda i: (i, 0)),
                   pl.BlockSpec((BLK, 16), lambda i: (i, 0)),
                   pl.BlockSpec((BLK, 16), lambda i: (i, 0))),
        out_shape=(jax.ShapeDtypeStruct((N, HID), jnp.float32),
                   jax.ShapeDtypeStruct((N, 16), jnp.float32),
                   jax.ShapeDtypeStruct((N, 16), jnp.float32)),
    )(stats, z, w, bnw, bnb, af_s, af_d)


def _gatfin_body(numa_ref, numb_ref, dena_ref, denb_ref, bias_ref,
                 o_ref, ost_ref):
    num = numa_ref[0] + numb_ref[0]
    den16 = dena_ref[0] + denb_ref[0]
    o = _finish_gat(num, den16, bias_ref[...])
    o_ref[...] = o
    @pl.when(pl.program_id(0) == 0)
    def _():
        ost_ref[...] = jnp.zeros_like(ost_ref)
    ost_ref[0:1, :] += jnp.sum(o, axis=0, keepdims=True)
    ost_ref[1:2, :] += jnp.sum(o * o, axis=0, keepdims=True)


def _gatfin_call(num2, den2, bias):
    full = lambda r: pl.BlockSpec((r, HID), lambda i: (0, 0))
    return pl.pallas_call(
        _gatfin_body,
        grid=(NBLK,),
        in_specs=[pl.BlockSpec((1, BLK, HID), lambda i: (0, i, 0)),
                  pl.BlockSpec((1, BLK, HID), lambda i: (1, i, 0)),
                  pl.BlockSpec((1, BLK, 16), lambda i: (0, i, 0)),
                  pl.BlockSpec((1, BLK, 16), lambda i: (1, i, 0)), full(1)],
        out_specs=(pl.BlockSpec((BLK, HID), lambda i: (i, 0)),
                   pl.BlockSpec((2, HID), lambda i: (0, 0))),
        out_shape=(jax.ShapeDtypeStruct((N, HID), jnp.float32),
                   jax.ShapeDtypeStruct((2, HID), jnp.float32)),
    )(num2, num2, den2, den2, bias)


def _pool_body(numa_ref, numb_ref, dena_ref, denb_ref, bias_ref, batch_ref,
               g_ref):
    num = numa_ref[0] + numb_ref[0]
    den16 = dena_ref[0] + denb_ref[0]
    o = _finish_gat(num, den16, bias_ref[...])
    gids = jax.lax.broadcasted_iota(jnp.int32, (NGRAPH, BLK), 0)
    m = (gids == batch_ref[0]).astype(jnp.float32)
    @pl.when(pl.program_id(0) == 0)
    def _():
        g_ref[...] = jnp.zeros_like(g_ref)
    g_ref[...] += _mm(m, o)


def _pool_call(num2, den2, bias, batch3d):
    full = lambda r: pl.BlockSpec((r, HID), lambda i: (0, 0))
    return pl.pallas_call(
        _pool_body,
        grid=(NBLK,),
        in_specs=[pl.BlockSpec((1, BLK, HID), lambda i: (0, i, 0)),
                  pl.BlockSpec((1, BLK, HID), lambda i: (1, i, 0)),
                  pl.BlockSpec((1, BLK, 16), lambda i: (0, i, 0)),
                  pl.BlockSpec((1, BLK, 16), lambda i: (1, i, 0)), full(1),
                  pl.BlockSpec((1, 1, BLK), lambda i: (i, 0, 0))],
        out_specs=pl.BlockSpec((NGRAPH, HID), lambda i: (0, 0)),
        out_shape=jax.ShapeDtypeStruct((NGRAPH, HID), jnp.float32),
    )(num2, num2, den2, den2, bias, batch3d)


def _edge_body(h_hbm, as_hbm, ad_hbm, src_hbm, dst_hbm, zrow_hbm, zden_hbm,
               num_out, den_out,
               num_acc, den_acc, src_v, dst_v, arows, brows, wrows, hrows,
               sems):
    cid = jax.lax.axis_index("c")
    sid = jax.lax.axis_index("s")
    base_row = sid * RPT
    pltpu.sync_copy(zrow_hbm, num_acc.at[pl.ds(base_row, RPT), :])
    pltpu.sync_copy(zden_hbm, den_acc.at[pl.ds(base_row, RPT), :])
    plsc.subcore_barrier()
    wid = cid * 16 + sid

    def issue(b, base):
        pltpu.sync_copy(src_hbm.at[pl.ds(base, C_E)], src_v.at[b])
        pltpu.sync_copy(dst_hbm.at[pl.ds(base, C_E)], dst_v.at[b])
        pltpu.async_copy(h_hbm.at[src_v.at[b]], hrows.at[b], sems.at[b, 0])
        pltpu.async_copy(as_hbm.at[src_v.at[b]], arows.at[b], sems.at[b, 1])
        pltpu.async_copy(ad_hbm.at[dst_v.at[b]], brows.at[b], sems.at[b, 2])

    def drain_scatter(b):
        pltpu.make_async_copy(wrows.at[b], den_acc.at[dst_v.at[b]],
                              sems.at[b, 3]).wait()
        pltpu.make_async_copy(hrows.at[b], num_acc.at[dst_v.at[b]],
                              sems.at[b, 4]).wait()

    def process(b):
        pltpu.make_async_copy(as_hbm.at[src_v.at[b]], arows.at[b],
                              sems.at[b, 1]).wait()
        pltpu.make_async_copy(ad_hbm.at[dst_v.at[b]], brows.at[b],
                              sems.at[b, 2]).wait()
        pltpu.make_async_copy(h_hbm.at[src_v.at[b]], hrows.at[b],
                              sems.at[b, 0]).wait()

        def eloop(u, c2):
            e0 = u * 4
            ws_all = []
            for du in range(4):
                e = e0 + du
                s16 = arows[b, e, :] + brows[b, e, :]
                w16 = jnp.exp(jnp.maximum(s16, 0.0)
                              + 0.2 * jnp.minimum(s16, 0.0))
                wrows[b, e, :] = w16
                ws_all.append([w16[hd] for hd in range(HEADS)])
            for du in range(4):
                e = e0 + du
                for hd in range(HEADS):
                    ws = ws_all[du][hd]
                    for q in range(2):
                        col = hd * OUT_H + q * 16
                        hrows[b, e, pl.ds(col, 16)] = (
                            hrows[b, e, pl.ds(col, 16)] * ws)
            return c2
        jax.lax.fori_loop(0, C_E // 4, eloop, 0)
        pltpu.async_copy(wrows.at[b], den_acc.at[dst_v.at[b]],
                         sems.at[b, 3], add=True)
        pltpu.async_copy(hrows.at[b], num_acc.at[dst_v.at[b]],
                         sems.at[b, 4], add=True)

    cbase = wid * EC * C_E
    issue(0, cbase)
    issue(1, cbase + C_E)

    def pair(j, carry):
        base_n = cbase + (2 * j + 2) * C_E
        process(0)
        @pl.when(j + 1 < EC // 2)
        def _():
            drain_scatter(0)
            issue(0, base_n)
        process(1)
        @pl.when(j + 1 < EC // 2)
        def _():
            drain_scatter(1)
            issue(1, base_n + C_E)
        return carry

    jax.lax.fori_loop(0, EC // 2, pair, 0)
    drain_scatter(0)
    drain_scatter(1)
    plsc.subcore_barrier()
    pltpu.sync_copy(num_acc.at[pl.ds(base_row, RPT), :],
                    num_out.at[cid, pl.ds(base_row, RPT), :])
    pltpu.sync_copy(den_acc.at[pl.ds(base_row, RPT), :],
                    den_out.at[cid, pl.ds(base_row, RPT), :])


_edge_call = pl.kernel(
    _edge_body,
    out_type=(jax.ShapeDtypeStruct((2, N_PAD, HID), jnp.float32),
              jax.ShapeDtypeStruct((2, N_PAD, 16), jnp.float32)),
    mesh=plsc.VectorSubcoreMesh(core_axis_name="c", subcore_axis_name="s"),
    compiler_params=pltpu.CompilerParams(use_tc_tiling_on_sc=False),
    scratch_types=[
        pltpu.VMEM_SHARED((N_PAD, HID), jnp.float32),
        pltpu.VMEM_SHARED((N_PAD, 16), jnp.float32),
        pltpu.VMEM((2, C_E), jnp.int32),
        pltpu.VMEM((2, C_E), jnp.int32),
        pltpu.VMEM((2, C_E, 16), jnp.float32),
        pltpu.VMEM((2, C_E, 16), jnp.float32),
        pltpu.VMEM((2, C_E, 16), jnp.float32),
        pltpu.VMEM((2, C_E, HID), jnp.float32),
        pltpu.SemaphoreType.DMA((2, 5)),
    ],
)


def _edge_aggregate(h_pad, as_pad, ad_pad, src_pad, dst_pad):
    zrow = jnp.zeros((RPT, HID), jnp.float32)
    zden = jnp.zeros((RPT, 16), jnp.float32)
    return _edge_call(h_pad, as_pad, ad_pad, src_pad, dst_pad, zrow, zden)


def _head_body(g_ref, bnw_ref, bnb_ref, wc_ref, bc_ref, out_ref):
    g = g_ref[...]
    m = jnp.mean(g, axis=0, keepdims=True)
    v = jnp.mean((g - m) ** 2, axis=0, keepdims=True)
    g = (g - m) * jax.lax.rsqrt(v + 1e-5) * bnw_ref[...] + bnb_ref[...]
    logits = _mm(g, wc_ref[...]) + bc_ref[...]
    mx = jnp.max(logits, axis=-1, keepdims=True)
    lse = mx + jnp.log(jnp.sum(jnp.exp(logits - mx), axis=-1, keepdims=True))
    out_ref[...] = logits - lse


def _head_call(g, bnw, bnb, wc, bc):
    return pl.pallas_call(
        _head_body,
        out_shape=jax.ShapeDtypeStruct((NGRAPH, NCLS), jnp.float32),
    )(g, bnw, bnb, wc, bc)


def _padn(a):
    return jnp.pad(a, ((0, N_PAD - N), (0, 0)))


def kernel(x, edge_index, batch, bn_feat_w, bn_feat_b, W0, b0,
           bn1_w, bn1_b, W1, a1s, a1d, bias1,
           bn2_w, bn2_b, W2, a2s, a2d, bias2,
           bn3_w, bn3_b, W3, a3s, a3d, bias3,
           bnh_w, bnh_b, Wc, bc):
    r1 = lambda v: v.reshape(1, -1)
    loops = jnp.arange(N, dtype=jnp.int32)
    padidx = jnp.full((E_PAD - E_RAW - N,), N, jnp.int32)
    src = jnp.concatenate([edge_index[0], loops, padidx])
    dst = jnp.concatenate([edge_index[1], loops, padidx])

    xst = _stats_call(x)
    z1, zst = _inproj_call(xst, x, W0, r1(b0), r1(bn_feat_w), r1(bn_feat_b))
    h1, as1, ad1 = _gatproj_call(zst, z1, W1, r1(bn1_w), r1(bn1_b),
                                 r1(a1s), r1(a1d))
    num1, den1 = _edge_aggregate(_padn(h1), _padn(as1), _padn(ad1), src, dst)

    o1, o1st = _gatfin_call(num1, den1, r1(bias1))
    h2, as2, ad2 = _gatproj_call(o1st, o1, W2, r1(bn2_w), r1(bn2_b),
                                 r1(a2s), r1(a2d))
    num2, den2 = _edge_aggregate(_padn(h2), _padn(as2), _padn(ad2), src, dst)

    o2, o2st = _gatfin_call(num2, den2, r1(bias2))
    h3, as3, ad3 = _gatproj_call(o2st, o2, W3, r1(bn3_w), r1(bn3_b),
                                 r1(a3s), r1(a3d))
    num3, den3 = _edge_aggregate(_padn(h3), _padn(as3), _padn(ad3), src, dst)

    g = _pool_call(num3, den3, r1(bias3), batch.reshape(NBLK, 1, BLK))
    return _head_call(g, r1(bnh_w), r1(bnh_b), Wc, r1(bc))


# speedup vs baseline: 1.2431x; 1.0451x over previous
import functools

import jax
import jax.numpy as jnp
from jax.experimental import pallas as pl
from jax.experimental.pallas import tpu as pltpu
from jax.experimental.pallas import tpu_sc as plsc

N = 10000
E_RAW = 320000
F_IN = 128
HID = 128
HEADS = 4
OUT_H = 32
NCLS = 16
NGRAPH = 64

BLK = 2000
NBLK = N // BLK

N_PAD = 10240
RPT = N_PAD // 16
C_E = 64
EC = 162
NW = 32
E_PAD = NW * EC * C_E

_PREC = jax.lax.Precision.HIGHEST


def _mm(a, b):
    return jax.lax.dot_general(a, b, (((1,), (0,)), ((), ())), precision=_PREC)


def _seg_matrix():
    r = jax.lax.broadcasted_iota(jnp.int32, (HID, 16), 0)
    c = jax.lax.broadcasted_iota(jnp.int32, (HID, 16), 1)
    return (c == r // OUT_H).astype(jnp.float32)


def _seg_matrix_t():
    r = jax.lax.broadcasted_iota(jnp.int32, (16, HID), 0)
    c = jax.lax.broadcasted_iota(jnp.int32, (16, HID), 1)
    return (r == c // OUT_H).astype(jnp.float32)


def _bn_scale_shift(stats_ref, w, b):
    m = stats_ref[0:1, :] * (1.0 / N)
    v = stats_ref[1:2, :] * (1.0 / N) - m * m
    scale = w * jax.lax.rsqrt(v + 1e-5)
    return scale, b - m * scale


def _finish_gat(num, den16, bias):
    denb = _mm(den16, _seg_matrix_t())
    return jnp.maximum(num / (denb + 1e-16) + bias, 0.0)


def _stats_body(x_ref, o_ref):
    @pl.when(pl.program_id(0) == 0)
    def _():
        o_ref[...] = jnp.zeros_like(o_ref)
    x = x_ref[...]
    o_ref[0:1, :] += jnp.sum(x, axis=0, keepdims=True)
    o_ref[1:2, :] += jnp.sum(x * x, axis=0, keepdims=True)


def _stats_call(x):
    return pl.pallas_call(
        _stats_body,
        grid=(NBLK,),
        in_specs=[pl.BlockSpec((BLK, HID), lambda i: (i, 0))],
        out_specs=pl.BlockSpec((2, HID), lambda i: (0, 0)),
        out_shape=jax.ShapeDtypeStruct((2, HID), jnp.float32),
    )(x)


def _inproj_body(stats_ref, x_ref, w0_ref, b0_ref, bnw_ref, bnb_ref,
                 z_ref, zst_ref):
    scale, shift = _bn_scale_shift(stats_ref, bnw_ref[...], bnb_ref[...])
    z = jnp.maximum(_mm(x_ref[...] * scale, w0_ref[...])
                    + _mm(shift, w0_ref[...]) + b0_ref[...], 0.0)
    z_ref[...] = z
    @pl.when(pl.program_id(0) == 0)
    def _():
        zst_ref[...] = jnp.zeros_like(zst_ref)
    zst_ref[0:1, :] += jnp.sum(z, axis=0, keepdims=True)
    zst_ref[1:2, :] += jnp.sum(z * z, axis=0, keepdims=True)


def _inproj_call(stats, x, w0, b0, bnw, bnb):
    full = lambda r: pl.BlockSpec((r, HID), lambda i: (0, 0))
    return pl.pallas_call(
        _inproj_body,
        grid=(NBLK,),
        in_specs=[full(2), pl.BlockSpec((BLK, HID), lambda i: (i, 0)),
                  full(HID), full(1), full(1), full(1)],
        out_specs=(pl.BlockSpec((BLK, HID), lambda i: (i, 0)),
                   pl.BlockSpec((2, HID), lambda i: (0, 0))),
        out_shape=(jax.ShapeDtypeStruct((N, HID), jnp.float32),
                   jax.ShapeDtypeStruct((2, HID), jnp.float32)),
    )(stats, x, w0, b0, bnw, bnb)


def _gatproj_body(stats_ref, z_ref, w_ref, bnw_ref, bnb_ref, af_s_ref,
                  af_d_ref, h_ref, as_ref, ad_ref):
    scale, shift = _bn_scale_shift(stats_ref, bnw_ref[...], bnb_ref[...])
    h = _mm(z_ref[...] * scale, w_ref[...]) + _mm(shift, w_ref[...])
    h_ref[...] = h
    s = _seg_matrix()
    as_ref[...] = _mm(h * af_s_ref[...], s)
    ad_ref[...] = _mm(h * af_d_ref[...], s)


def _gatproj_call(stats, z, w, bnw, bnb, af_s, af_d):
    full = lambda r: pl.BlockSpec((r, HID), lambda i: (0, 0))
    return pl.pallas_call(
        _gatproj_body,
        grid=(NBLK,),
        in_specs=[full(2), pl.BlockSpec((BLK, HID), lambda i: (i, 0)),
                  full(HID), full(1), full(1), full(1), full(1)],
        out_specs=(pl.BlockSpec((BLK, HID), lambda i: (i, 0)),
                   pl.BlockSpec((BLK, 16), lambda i: (i, 0)),
                   pl.BlockSpec((BLK, 16), lambda i: (i, 0))),
        out_shape=(jax.ShapeDtypeStruct((N, HID), jnp.float32),
                   jax.ShapeDtypeStruct((N, 16), jnp.float32),
                   jax.ShapeDtypeStruct((N, 16), jnp.float32)),
    )(stats, z, w, bnw, bnb, af_s, af_d)


def _gatfin_body(numa_ref, numb_ref, dena_ref, denb_ref, bias_ref,
                 o_ref, ost_ref):
    num = numa_ref[0] + numb_ref[0]
    den16 = dena_ref[0] + denb_ref[0]
    o = _finish_gat(num, den16, bias_ref[...])
    o_ref[...] = o
    @pl.when(pl.program_id(0) == 0)
    def _():
        ost_ref[...] = jnp.zeros_like(ost_ref)
    ost_ref[0:1, :] += jnp.sum(o, axis=0, keepdims=True)
    ost_ref[1:2, :] += jnp.sum(o * o, axis=0, keepdims=True)


def _gatfin_call(num2, den2, bias):
    full = lambda r: pl.BlockSpec((r, HID), lambda i: (0, 0))
    return pl.pallas_call(
        _gatfin_body,
        grid=(NBLK,),
        in_specs=[pl.BlockSpec((1, BLK, HID), lambda i: (0, i, 0)),
                  pl.BlockSpec((1, BLK, HID), lambda i: (1, i, 0)),
                  pl.BlockSpec((1, BLK, 16), lambda i: (0, i, 0)),
                  pl.BlockSpec((1, BLK, 16), lambda i: (1, i, 0)), full(1)],
        out_specs=(pl.BlockSpec((BLK, HID), lambda i: (i, 0)),
                   pl.BlockSpec((2, HID), lambda i: (0, 0))),
        out_shape=(jax.ShapeDtypeStruct((N, HID), jnp.float32),
                   jax.ShapeDtypeStruct((2, HID), jnp.float32)),
    )(num2, num2, den2, den2, bias)


def _pool_body(numa_ref, numb_ref, dena_ref, denb_ref, bias_ref, batch_ref,
               g_ref):
    num = numa_ref[0] + numb_ref[0]
    den16 = dena_ref[0] + denb_ref[0]
    o = _finish_gat(num, den16, bias_ref[...])
    gids = jax.lax.broadcasted_iota(jnp.int32, (NGRAPH, BLK), 0)
    m = (gids == batch_ref[0]).astype(jnp.float32)
    @pl.when(pl.program_id(0) == 0)
    def _():
        g_ref[...] = jnp.zeros_like(g_ref)
    g_ref[...] += _mm(m, o)


def _pool_call(num2, den2, bias, batch3d):
    full = lambda r: pl.BlockSpec((r, HID), lambda i: (0, 0))
    return pl.pallas_call(
        _pool_body,
        grid=(NBLK,),
        in_specs=[pl.BlockSpec((1, BLK, HID), lambda i: (0, i, 0)),
                  pl.BlockSpec((1, BLK, HID), lambda i: (1, i, 0)),
                  pl.BlockSpec((1, BLK, 16), lambda i: (0, i, 0)),
                  pl.BlockSpec((1, BLK, 16), lambda i: (1, i, 0)), full(1),
                  pl.BlockSpec((1, 1, BLK), lambda i: (i, 0, 0))],
        out_specs=pl.BlockSpec((NGRAPH, HID), lambda i: (0, 0)),
        out_shape=jax.ShapeDtypeStruct((NGRAPH, HID), jnp.float32),
    )(num2, num2, den2, den2, bias, batch3d)


def _edge_body(h_hbm, as_hbm, ad_hbm, src_hbm, dst_hbm, zrow_hbm, zden_hbm,
               num_out, den_out,
               num_acc, den_acc, src_v, dst_v, arows, brows, wrows, hrows,
               sems):
    cid = jax.lax.axis_index("c")
    sid = jax.lax.axis_index("s")
    base_row = sid * RPT
    pltpu.sync_copy(zrow_hbm, num_acc.at[pl.ds(base_row, RPT), :])
    pltpu.sync_copy(zden_hbm, den_acc.at[pl.ds(base_row, RPT), :])
    plsc.subcore_barrier()
    wid = cid * 16 + sid

    def issue(b, base):
        pltpu.sync_copy(src_hbm.at[pl.ds(base, C_E)], src_v.at[b])
        pltpu.sync_copy(dst_hbm.at[pl.ds(base, C_E)], dst_v.at[b])
        pltpu.async_copy(h_hbm.at[src_v.at[b]], hrows.at[b], sems.at[b, 0])
        pltpu.async_copy(as_hbm.at[src_v.at[b]], arows.at[b], sems.at[b, 1])
        pltpu.async_copy(ad_hbm.at[dst_v.at[b]], brows.at[b], sems.at[b, 2])

    def drain_scatter(b):
        pltpu.make_async_copy(wrows.at[b], den_acc.at[dst_v.at[b]],
                              sems.at[b, 3]).wait()
        pltpu.make_async_copy(hrows.at[b], num_acc.at[dst_v.at[b]],
                              sems.at[b, 4]).wait()

    def process(b):
        pltpu.make_async_copy(as_hbm.at[src_v.at[b]], arows.at[b],
                              sems.at[b, 1]).wait()
        pltpu.make_async_copy(ad_hbm.at[dst_v.at[b]], brows.at[b],
                              sems.at[b, 2]).wait()
        pltpu.make_async_copy(h_hbm.at[src_v.at[b]], hrows.at[b],
                              sems.at[b, 0]).wait()

        def eloop(u, c2):
            e0 = u * 8
            ws_all = []
            for du in range(8):
                e = e0 + du
                s16 = arows[b, e, :] + brows[b, e, :]
                w16 = jnp.exp(jnp.maximum(s16, 0.0)
                              + 0.2 * jnp.minimum(s16, 0.0))
                wrows[b, e, :] = w16
                ws_all.append([w16[hd] for hd in range(HEADS)])
            for du in range(8):
                e = e0 + du
                for hd in range(HEADS):
                    ws = ws_all[du][hd]
                    for q in range(2):
                        col = hd * OUT_H + q * 16
                        hrows[b, e, pl.ds(col, 16)] = (
                            hrows[b, e, pl.ds(col, 16)] * ws)
            return c2
        jax.lax.fori_loop(0, C_E // 8, eloop, 0)
        pltpu.async_copy(wrows.at[b], den_acc.at[dst_v.at[b]],
                         sems.at[b, 3], add=True)
        pltpu.async_copy(hrows.at[b], num_acc.at[dst_v.at[b]],
                         sems.at[b, 4], add=True)

    cbase = wid * EC * C_E
    issue(0, cbase)
    issue(1, cbase + C_E)

    def pair(j, carry):
        base_n = cbase + (2 * j + 2) * C_E
        process(0)
        @pl.when(j + 1 < EC // 2)
        def _():
            drain_scatter(0)
            issue(0, base_n)
        process(1)
        @pl.when(j + 1 < EC // 2)
        def _():
            drain_scatter(1)
            issue(1, base_n + C_E)
        return carry

    jax.lax.fori_loop(0, EC // 2, pair, 0)
    drain_scatter(0)
    drain_scatter(1)
    plsc.subcore_barrier()
    pltpu.sync_copy(num_acc.at[pl.ds(base_row, RPT), :],
                    num_out.at[cid, pl.ds(base_row, RPT), :])
    pltpu.sync_copy(den_acc.at[pl.ds(base_row, RPT), :],
                    den_out.at[cid, pl.ds(base_row, RPT), :])


_edge_call = pl.kernel(
    _edge_body,
    out_type=(jax.ShapeDtypeStruct((2, N_PAD, HID), jnp.float32),
              jax.ShapeDtypeStruct((2, N_PAD, 16), jnp.float32)),
    mesh=plsc.VectorSubcoreMesh(core_axis_name="c", subcore_axis_name="s"),
    compiler_params=pltpu.CompilerParams(use_tc_tiling_on_sc=False),
    scratch_types=[
        pltpu.VMEM_SHARED((N_PAD, HID), jnp.float32),
        pltpu.VMEM_SHARED((N_PAD, 16), jnp.float32),
        pltpu.VMEM((2, C_E), jnp.int32),
        pltpu.VMEM((2, C_E), jnp.int32),
        pltpu.VMEM((2, C_E, 16), jnp.float32),
        pltpu.VMEM((2, C_E, 16), jnp.float32),
        pltpu.VMEM((2, C_E, 16), jnp.float32),
        pltpu.VMEM((2, C_E, HID), jnp.float32),
        pltpu.SemaphoreType.DMA((2, 5)),
    ],
)


def _edge_aggregate(h_pad, as_pad, ad_pad, src_pad, dst_pad):
    zrow = jnp.zeros((RPT, HID), jnp.float32)
    zden = jnp.zeros((RPT, 16), jnp.float32)
    return _edge_call(h_pad, as_pad, ad_pad, src_pad, dst_pad, zrow, zden)


def _head_body(g_ref, bnw_ref, bnb_ref, wc_ref, bc_ref, out_ref):
    g = g_ref[...]
    m = jnp.mean(g, axis=0, keepdims=True)
    v = jnp.mean((g - m) ** 2, axis=0, keepdims=True)
    g = (g - m) * jax.lax.rsqrt(v + 1e-5) * bnw_ref[...] + bnb_ref[...]
    logits = _mm(g, wc_ref[...]) + bc_ref[...]
    mx = jnp.max(logits, axis=-1, keepdims=True)
    lse = mx + jnp.log(jnp.sum(jnp.exp(logits - mx), axis=-1, keepdims=True))
    out_ref[...] = logits - lse


def _head_call(g, bnw, bnb, wc, bc):
    return pl.pallas_call(
        _head_body,
        out_shape=jax.ShapeDtypeStruct((NGRAPH, NCLS), jnp.float32),
    )(g, bnw, bnb, wc, bc)


def _padn(a):
    return jnp.pad(a, ((0, N_PAD - N), (0, 0)))


def kernel(x, edge_index, batch, bn_feat_w, bn_feat_b, W0, b0,
           bn1_w, bn1_b, W1, a1s, a1d, bias1,
           bn2_w, bn2_b, W2, a2s, a2d, bias2,
           bn3_w, bn3_b, W3, a3s, a3d, bias3,
           bnh_w, bnh_b, Wc, bc):
    r1 = lambda v: v.reshape(1, -1)
    loops = jnp.arange(N, dtype=jnp.int32)
    padidx = jnp.full((E_PAD - E_RAW - N,), N, jnp.int32)
    src = jnp.concatenate([edge_index[0], loops, padidx])
    dst = jnp.concatenate([edge_index[1], loops, padidx])

    xst = _stats_call(x)
    z1, zst = _inproj_call(xst, x, W0, r1(b0), r1(bn_feat_w), r1(bn_feat_b))
    h1, as1, ad1 = _gatproj_call(zst, z1, W1, r1(bn1_w), r1(bn1_b),
                                 r1(a1s), r1(a1d))
    num1, den1 = _edge_aggregate(_padn(h1), _padn(as1), _padn(ad1), src, dst)

    o1, o1st = _gatfin_call(num1, den1, r1(bias1))
    h2, as2, ad2 = _gatproj_call(o1st, o1, W2, r1(bn2_w), r1(bn2_b),
                                 r1(a2s), r1(a2d))
    num2, den2 = _edge_aggregate(_padn(h2), _padn(as2), _padn(ad2), src, dst)

    o2, o2st = _gatfin_call(num2, den2, r1(bias2))
    h3, as3, ad3 = _gatproj_call(o2st, o2, W3, r1(bn3_w), r1(bn3_b),
                                 r1(a3s), r1(a3d))
    num3, den3 = _edge_aggregate(_padn(h3), _padn(as3), _padn(ad3), src, dst)

    g = _pool_call(num3, den3, r1(bias3), batch.reshape(NBLK, 1, BLK))
    return _head_call(g, r1(bnh_w), r1(bnh_b), Wc, r1(bc))


# speedup vs baseline: 1.2432x; 1.0001x over previous
import functools

import jax
import jax.numpy as jnp
from jax.experimental import pallas as pl
from jax.experimental.pallas import tpu as pltpu
from jax.experimental.pallas import tpu_sc as plsc

N = 10000
E_RAW = 320000
F_IN = 128
HID = 128
HEADS = 4
OUT_H = 32
NCLS = 16
NGRAPH = 64

BLK = 2000
NBLK = N // BLK

N_PAD = 10240
RPT = N_PAD // 16
C_E = 64
EC = 162
NW = 32
E_PAD = NW * EC * C_E

_PREC = jax.lax.Precision.HIGHEST


def _mm(a, b):
    return jax.lax.dot_general(a, b, (((1,), (0,)), ((), ())), precision=_PREC)


def _seg_matrix():
    r = jax.lax.broadcasted_iota(jnp.int32, (HID, 16), 0)
    c = jax.lax.broadcasted_iota(jnp.int32, (HID, 16), 1)
    return (c == r // OUT_H).astype(jnp.float32)


def _seg_matrix_t():
    r = jax.lax.broadcasted_iota(jnp.int32, (16, HID), 0)
    c = jax.lax.broadcasted_iota(jnp.int32, (16, HID), 1)
    return (r == c // OUT_H).astype(jnp.float32)


def _bn_scale_shift(stats_ref, w, b):
    m = stats_ref[0:1, :] * (1.0 / N)
    v = stats_ref[1:2, :] * (1.0 / N) - m * m
    scale = w * jax.lax.rsqrt(v + 1e-5)
    return scale, b - m * scale


def _finish_gat(num, den16, bias):
    denb = _mm(den16, _seg_matrix_t())
    return jnp.maximum(num / (denb + 1e-16) + bias, 0.0)


def _stats_body(x_ref, o_ref):
    @pl.when(pl.program_id(0) == 0)
    def _():
        o_ref[...] = jnp.zeros_like(o_ref)
    x = x_ref[...]
    o_ref[0:1, :] += jnp.sum(x, axis=0, keepdims=True)
    o_ref[1:2, :] += jnp.sum(x * x, axis=0, keepdims=True)


def _stats_call(x):
    return pl.pallas_call(
        _stats_body,
        grid=(NBLK,),
        in_specs=[pl.BlockSpec((BLK, HID), lambda i: (i, 0))],
        out_specs=pl.BlockSpec((2, HID), lambda i: (0, 0)),
        out_shape=jax.ShapeDtypeStruct((2, HID), jnp.float32),
    )(x)


def _inproj_body(stats_ref, x_ref, w0_ref, b0_ref, bnw_ref, bnb_ref,
                 z_ref, zst_ref):
    scale, shift = _bn_scale_shift(stats_ref, bnw_ref[...], bnb_ref[...])
    z = jnp.maximum(_mm(x_ref[...] * scale, w0_ref[...])
                    + _mm(shift, w0_ref[...]) + b0_ref[...], 0.0)
    z_ref[...] = z
    @pl.when(pl.program_id(0) == 0)
    def _():
        zst_ref[...] = jnp.zeros_like(zst_ref)
    zst_ref[0:1, :] += jnp.sum(z, axis=0, keepdims=True)
    zst_ref[1:2, :] += jnp.sum(z * z, axis=0, keepdims=True)


def _inproj_call(stats, x, w0, b0, bnw, bnb):
    full = lambda r: pl.BlockSpec((r, HID), lambda i: (0, 0))
    return pl.pallas_call(
        _inproj_body,
        grid=(NBLK,),
        in_specs=[full(2), pl.BlockSpec((BLK, HID), lambda i: (i, 0)),
                  full(HID), full(1), full(1), full(1)],
        out_specs=(pl.BlockSpec((BLK, HID), lambda i: (i, 0)),
                   pl.BlockSpec((2, HID), lambda i: (0, 0))),
        out_shape=(jax.ShapeDtypeStruct((N, HID), jnp.float32),
                   jax.ShapeDtypeStruct((2, HID), jnp.float32)),
    )(stats, x, w0, b0, bnw, bnb)


def _gatproj_body(stats_ref, z_ref, w_ref, bnw_ref, bnb_ref, af_s_ref,
                  af_d_ref, h_ref, as_ref, ad_ref):
    scale, shift = _bn_scale_shift(stats_ref, bnw_ref[...], bnb_ref[...])
    h = _mm(z_ref[...] * scale, w_ref[...]) + _mm(shift, w_ref[...])
    h_ref[...] = h
    s = _seg_matrix()
    as_ref[...] = _mm(h * af_s_ref[...], s)
    ad_ref[...] = _mm(h * af_d_ref[...], s)


def _gatproj_call(stats, z, w, bnw, bnb, af_s, af_d):
    full = lambda r: pl.BlockSpec((r, HID), lambda i: (0, 0))
    return pl.pallas_call(
        _gatproj_body,
        grid=(NBLK,),
        in_specs=[full(2), pl.BlockSpec((BLK, HID), lambda i: (i, 0)),
                  full(HID), full(1), full(1), full(1), full(1)],
        out_specs=(pl.BlockSpec((BLK, HID), lambda i: (i, 0)),
                   pl.BlockSpec((BLK, 16), lambda i: (i, 0)),
                   pl.BlockSpec((BLK, 16), lambda i: (i, 0))),
        out_shape=(jax.ShapeDtypeStruct((N, HID), jnp.float32),
                   jax.ShapeDtypeStruct((N, 16), jnp.float32),
                   jax.ShapeDtypeStruct((N, 16), jnp.float32)),
    )(stats, z, w, bnw, bnb, af_s, af_d)


def _gatfin_body(numa_ref, numb_ref, dena_ref, denb_ref, bias_ref,
                 o_ref, ost_ref):
    num = numa_ref[0] + numb_ref[0]
    den16 = dena_ref[0] + denb_ref[0]
    o = _finish_gat(num, den16, bias_ref[...])
    o_ref[...] = o
    @pl.when(pl.program_id(0) == 0)
    def _():
        ost_ref[...] = jnp.zeros_like(ost_ref)
    ost_ref[0:1, :] += jnp.sum(o, axis=0, keepdims=True)
    ost_ref[1:2, :] += jnp.sum(o * o, axis=0, keepdims=True)


def _gatfin_call(num2, den2, bias):
    full = lambda r: pl.BlockSpec((r, HID), lambda i: (0, 0))
    return pl.pallas_call(
        _gatfin_body,
        grid=(NBLK,),
        in_specs=[pl.BlockSpec((1, BLK, HID), lambda i: (0, i, 0)),
                  pl.BlockSpec((1, BLK, HID), lambda i: (1, i, 0)),
                  pl.BlockSpec((1, BLK, 16), lambda i: (0, i, 0)),
                  pl.BlockSpec((1, BLK, 16), lambda i: (1, i, 0)), full(1)],
        out_specs=(pl.BlockSpec((BLK, HID), lambda i: (i, 0)),
                   pl.BlockSpec((2, HID), lambda i: (0, 0))),
        out_shape=(jax.ShapeDtypeStruct((N, HID), jnp.float32),
                   jax.ShapeDtypeStruct((2, HID), jnp.float32)),
    )(num2, num2, den2, den2, bias)


def _pool_body(numa_ref, numb_ref, dena_ref, denb_ref, bias_ref, batch_ref,
               g_ref):
    num = numa_ref[0] + numb_ref[0]
    den16 = dena_ref[0] + denb_ref[0]
    o = _finish_gat(num, den16, bias_ref[...])
    gids = jax.lax.broadcasted_iota(jnp.int32, (NGRAPH, BLK), 0)
    m = (gids == batch_ref[0]).astype(jnp.float32)
    @pl.when(pl.program_id(0) == 0)
    def _():
        g_ref[...] = jnp.zeros_like(g_ref)
    g_ref[...] += _mm(m, o)


def _pool_call(num2, den2, bias, batch3d):
    full = lambda r: pl.BlockSpec((r, HID), lambda i: (0, 0))
    return pl.pallas_call(
        _pool_body,
        grid=(NBLK,),
        in_specs=[pl.BlockSpec((1, BLK, HID), lambda i: (0, i, 0)),
                  pl.BlockSpec((1, BLK, HID), lambda i: (1, i, 0)),
                  pl.BlockSpec((1, BLK, 16), lambda i: (0, i, 0)),
                  pl.BlockSpec((1, BLK, 16), lambda i: (1, i, 0)), full(1),
                  pl.BlockSpec((1, 1, BLK), lambda i: (i, 0, 0))],
        out_specs=pl.BlockSpec((NGRAPH, HID), lambda i: (0, 0)),
        out_shape=jax.ShapeDtypeStruct((NGRAPH, HID), jnp.float32),
    )(num2, num2, den2, den2, bias, batch3d)


def _edge_body(h_hbm, as_hbm, ad_hbm, src_hbm, dst_hbm, zrow_hbm, zden_hbm,
               num_out, den_out,
               num_acc, den_acc, src_v, dst_v, arows, brows, wrows, hrows,
               sems):
    cid = jax.lax.axis_index("c")
    sid = jax.lax.axis_index("s")
    base_row = sid * RPT
    pltpu.sync_copy(zrow_hbm, num_acc.at[pl.ds(base_row, RPT), :])
    pltpu.sync_copy(zden_hbm, den_acc.at[pl.ds(base_row, RPT), :])
    plsc.subcore_barrier()
    wid = cid * 16 + sid

    def issue(b, base):
        pltpu.sync_copy(src_hbm.at[pl.ds(base, C_E)], src_v.at[b])
        pltpu.sync_copy(dst_hbm.at[pl.ds(base, C_E)], dst_v.at[b])
        pltpu.async_copy(h_hbm.at[src_v.at[b]], hrows.at[b], sems.at[b, 0])
        pltpu.async_copy(as_hbm.at[src_v.at[b]], arows.at[b], sems.at[b, 1])
        pltpu.async_copy(ad_hbm.at[dst_v.at[b]], brows.at[b], sems.at[b, 2])

    def drain_scatter(b):
        pltpu.make_async_copy(wrows.at[b], den_acc.at[dst_v.at[b]],
                              sems.at[b, 3]).wait()
        pltpu.make_async_copy(hrows.at[b], num_acc.at[dst_v.at[b]],
                              sems.at[b, 4]).wait()

    def process(b):
        pltpu.make_async_copy(as_hbm.at[src_v.at[b]], arows.at[b],
                              sems.at[b, 1]).wait()
        pltpu.make_async_copy(ad_hbm.at[dst_v.at[b]], brows.at[b],
                              sems.at[b, 2]).wait()
        pltpu.make_async_copy(h_hbm.at[src_v.at[b]], hrows.at[b],
                              sems.at[b, 0]).wait()

        def eloop(u, c2):
            e0 = u * 8
            ws_all = []
            for du in range(8):
                e = e0 + du
                s16 = arows[b, e, :] + brows[b, e, :]
                w16 = jnp.exp(jnp.maximum(s16, 0.0)
                              + 0.2 * jnp.minimum(s16, 0.0))
                wrows[b, e, :] = w16
                ws_all.append([w16[hd] for hd in range(HEADS)])
            for du in range(8):
                e = e0 + du
                for hd in range(HEADS):
                    ws = ws_all[du][hd]
                    for q in range(2):
                        col = hd * OUT_H + q * 16
                        hrows[b, e, pl.ds(col, 16)] = (
                            hrows[b, e, pl.ds(col, 16)] * ws)
            return c2
        jax.lax.fori_loop(0, C_E // 8, eloop, 0)
        pltpu.async_copy(wrows.at[b], den_acc.at[dst_v.at[b]],
                         sems.at[b, 3], add=True)
        pltpu.async_copy(hrows.at[b], num_acc.at[dst_v.at[b]],
                         sems.at[b, 4], add=True)

    cbase = wid * EC * C_E
    issue(0, cbase)
    issue(1, cbase + C_E)

    def pair(j, carry):
        base_n = cbase + (2 * j + 2) * C_E
        process(0)
        @pl.when(j + 1 < EC // 2)
        def _():
            drain_scatter(0)
            issue(0, base_n)
        process(1)
        @pl.when(j + 1 < EC // 2)
        def _():
            drain_scatter(1)
            issue(1, base_n + C_E)
        return carry

    jax.lax.fori_loop(0, EC // 2, pair, 0)
    drain_scatter(0)
    drain_scatter(1)
    plsc.subcore_barrier()
    pltpu.sync_copy(num_acc.at[pl.ds(base_row, RPT), :],
                    num_out.at[cid, pl.ds(base_row, RPT), :])
    pltpu.sync_copy(den_acc.at[pl.ds(base_row, RPT), :],
                    den_out.at[cid, pl.ds(base_row, RPT), :])


@functools.lru_cache(maxsize=None)
def _edge_call():
    return pl.kernel(
        _edge_body,
        out_type=(jax.ShapeDtypeStruct((2, N_PAD, HID), jnp.float32),
                  jax.ShapeDtypeStruct((2, N_PAD, 16), jnp.float32)),
        mesh=plsc.VectorSubcoreMesh(core_axis_name="c",
                                    subcore_axis_name="s"),
        compiler_params=pltpu.CompilerParams(use_tc_tiling_on_sc=False),
        scratch_types=[
            pltpu.VMEM_SHARED((N_PAD, HID), jnp.float32),
            pltpu.VMEM_SHARED((N_PAD, 16), jnp.float32),
            pltpu.VMEM((2, C_E), jnp.int32),
            pltpu.VMEM((2, C_E), jnp.int32),
            pltpu.VMEM((2, C_E, 16), jnp.float32),
            pltpu.VMEM((2, C_E, 16), jnp.float32),
            pltpu.VMEM((2, C_E, 16), jnp.float32),
            pltpu.VMEM((2, C_E, HID), jnp.float32),
            pltpu.SemaphoreType.DMA((2, 5)),
        ],
    )


def _edge_aggregate(h_pad, as_pad, ad_pad, src_pad, dst_pad):
    zrow = jnp.zeros((RPT, HID), jnp.float32)
    zden = jnp.zeros((RPT, 16), jnp.float32)
    return _edge_call()(h_pad, as_pad, ad_pad, src_pad, dst_pad, zrow, zden)


def _head_body(g_ref, bnw_ref, bnb_ref, wc_ref, bc_ref, out_ref):
    g = g_ref[...]
    m = jnp.mean(g, axis=0, keepdims=True)
    v = jnp.mean((g - m) ** 2, axis=0, keepdims=True)
    g = (g - m) * jax.lax.rsqrt(v + 1e-5) * bnw_ref[...] + bnb_ref[...]
    logits = _mm(g, wc_ref[...]) + bc_ref[...]
    mx = jnp.max(logits, axis=-1, keepdims=True)
    lse = mx + jnp.log(jnp.sum(jnp.exp(logits - mx), axis=-1, keepdims=True))
    out_ref[...] = logits - lse


def _head_call(g, bnw, bnb, wc, bc):
    return pl.pallas_call(
        _head_body,
        out_shape=jax.ShapeDtypeStruct((NGRAPH, NCLS), jnp.float32),
    )(g, bnw, bnb, wc, bc)


def _padn(a):
    return jnp.pad(a, ((0, N_PAD - N), (0, 0)))


def kernel(x, edge_index, batch, bn_feat_w, bn_feat_b, W0, b0,
           bn1_w, bn1_b, W1, a1s, a1d, bias1,
           bn2_w, bn2_b, W2, a2s, a2d, bias2,
           bn3_w, bn3_b, W3, a3s, a3d, bias3,
           bnh_w, bnh_b, Wc, bc):
    r1 = lambda v: v.reshape(1, -1)
    loops = jnp.arange(N, dtype=jnp.int32)
    padidx = jnp.full((E_PAD - E_RAW - N,), N, jnp.int32)
    src = jnp.concatenate([edge_index[0], loops, padidx])
    dst = jnp.concatenate([edge_index[1], loops, padidx])

    xst = _stats_call(x)
    z1, zst = _inproj_call(xst, x, W0, r1(b0), r1(bn_feat_w), r1(bn_feat_b))
    h1, as1, ad1 = _gatproj_call(zst, z1, W1, r1(bn1_w), r1(bn1_b),
                                 r1(a1s), r1(a1d))
    num1, den1 = _edge_aggregate(_padn(h1), _padn(as1), _padn(ad1), src, dst)

    o1, o1st = _gatfin_call(num1, den1, r1(bias1))
    h2, as2, ad2 = _gatproj_call(o1st, o1, W2, r1(bn2_w), r1(bn2_b),
                                 r1(a2s), r1(a2d))
    num2, den2 = _edge_aggregate(_padn(h2), _padn(as2), _padn(ad2), src, dst)

    o2, o2st = _gatfin_call(num2, den2, r1(bias2))
    h3, as3, ad3 = _gatproj_call(o2st, o2, W3, r1(bn3_w), r1(bn3_b),
                                 r1(a3s), r1(a3d))
    num3, den3 = _edge_aggregate(_padn(h3), _padn(as3), _padn(ad3), src, dst)

    g = _pool_call(num3, den3, r1(bias3), batch.reshape(NBLK, 1, BLK))
    return _head_call(g, r1(bnh_w), r1(bnh_b), Wc, r1(bc))
